# Initial kernel scaffold; baseline (speedup 1.0000x reference)
import jax
import jax.numpy as jnp
from jax.experimental import pallas as pl


def kernel(x, edge_index, W1, b1, W2, b2):
    raise NotImplementedError("write your pallas kernel here")


# speedup vs baseline: 2.5489x; 2.5489x over previous
import functools

import jax
import jax.numpy as jnp
from jax import lax
from jax.experimental import pallas as pl
from jax.experimental.pallas import tpu as pltpu
from jax.experimental.pallas import tpu_sc as plsc

N_NODES = 10000
N_EDGES = 160000
D = 256
DH = 128

NS = 16
BATCH = 128
NB = 80
E_PAD = NS * NB * BATCH
ACC_ROWS = 10240
ZROWS = ACC_ROWS // NS
WB_CHUNK = 128

RB = 1000


def _mm1_body(x_ref, w_ref, o_ref):
    h = jnp.dot(x_ref[...], w_ref[...], preferred_element_type=jnp.float32)
    o_ref[0] = h[:, :DH]
    o_ref[1] = h[:, DH:]


def _mm1(x, w):
    return pl.pallas_call(
        _mm1_body,
        grid=(N_NODES // RB,),
        in_specs=[
            pl.BlockSpec((RB, D), lambda r: (r, 0)),
            pl.BlockSpec((D, D), lambda r: (0, 0)),
        ],
        out_specs=pl.BlockSpec((2, RB, DH), lambda r: (0, r, 0)),
        out_shape=jax.ShapeDtypeStruct((2, N_NODES, DH), jnp.float32),
    )(x, w)


def _mm2_body(s_ref, b_ref, w_ref, o_ref):
    s = jnp.concatenate([s_ref[0], s_ref[1]], axis=1)
    a = jnp.maximum(s + b_ref[...], 0.0)
    h = jnp.dot(a, w_ref[...], preferred_element_type=jnp.float32)
    o_ref[0] = h[:, :DH]
    o_ref[1] = h[:, DH:]


def _mm2(s, b, w):
    return pl.pallas_call(
        _mm2_body,
        grid=(N_NODES // RB,),
        in_specs=[
            pl.BlockSpec((2, RB, DH), lambda r: (0, r, 0)),
            pl.BlockSpec((1, D), lambda r: (0, 0)),
            pl.BlockSpec((D, D), lambda r: (0, 0)),
        ],
        out_specs=pl.BlockSpec((2, RB, DH), lambda r: (0, r, 0)),
        out_shape=jax.ShapeDtypeStruct((2, N_NODES, DH), jnp.float32),
    )(s, b, w)


def _fin_body(s_ref, b_ref, o_ref):
    s = jnp.concatenate([s_ref[0], s_ref[1]], axis=1)
    o_ref[...] = jnp.maximum(s + b_ref[...], 0.0)


def _fin(s, b):
    return pl.pallas_call(
        _fin_body,
        grid=(N_NODES // RB,),
        in_specs=[
            pl.BlockSpec((2, RB, DH), lambda r: (0, r, 0)),
            pl.BlockSpec((1, D), lambda r: (0, 0)),
        ],
        out_specs=pl.BlockSpec((RB, D), lambda r: (r, 0)),
        out_shape=jax.ShapeDtypeStruct((N_NODES, D), jnp.float32),
    )(s, b)


def _sc_scatter_body(arr_hbm, h_hbm, z_hbm, o_hbm, idx_v, gbuf, acc_sh, sem):
    c = lax.axis_index("c")
    s = lax.axis_index("s")

    pltpu.sync_copy(z_hbm, acc_sh.at[pl.ds(s * ZROWS, ZROWS)])
    plsc.subcore_barrier()

    h_c = h_hbm.at[c]

    @pl.loop(0, NB)
    def _edges(b):
        pltpu.sync_copy(arr_hbm.at[s, b], idx_v)
        pltpu.async_copy(h_c.at[idx_v.at[0]], gbuf, sem).wait()
        pltpu.sync_copy(gbuf, acc_sh.at[idx_v.at[1]], add=True)

    plsc.subcore_barrier()

    o_c = o_hbm.at[c]

    @pl.loop(0, ZROWS // WB_CHUNK)
    def _wb(k):
        base = s * ZROWS + k * WB_CHUNK
        pltpu.sync_copy(acc_sh.at[pl.ds(base, WB_CHUNK)], gbuf)
        pltpu.sync_copy(gbuf, o_c.at[pl.ds(base, WB_CHUNK)])


@functools.cache
def _sc_scatter_kernel():
    mesh = plsc.VectorSubcoreMesh(core_axis_name="c", subcore_axis_name="s")
    return pl.kernel(
        _sc_scatter_body,
        out_type=jax.ShapeDtypeStruct((2, ACC_ROWS, DH), jnp.float32),
        mesh=mesh,
        scratch_types=[
            pltpu.VMEM((2, BATCH), jnp.int32),
            pltpu.VMEM((BATCH, DH), jnp.float32),
            pltpu.VMEM_SHARED((ACC_ROWS, DH), jnp.float32),
            pltpu.SemaphoreType.DMA,
        ],
    )


def _sc_scatter(arr, h, zeros):
    return _sc_scatter_kernel()(arr, h, zeros)


def _prep_edges(edge_index):
    ei = edge_index.astype(jnp.int32)
    npad = E_PAD - N_EDGES
    col = jnp.concatenate([ei[1], jnp.zeros((npad,), jnp.int32)])
    row = jnp.concatenate([ei[0], jnp.full((npad,), N_NODES, jnp.int32)])
    return jnp.stack(
        [col.reshape(NS, NB, BATCH), row.reshape(NS, NB, BATCH)], axis=2)


def kernel(x, edge_index, W1, b1, W2, b2):
    arr = _prep_edges(edge_index)
    zeros = jnp.zeros((ZROWS, DH), jnp.float32)
    h1 = _mm1(x, W1)
    s1 = _sc_scatter(arr, h1, zeros)
    h2 = _mm2(s1, b1.reshape(1, D), W2)
    s2 = _sc_scatter(arr, h2, zeros)
    logits = _fin(s2, b2.reshape(1, D))
    return (logits, jnp.float32(0.0))


# speedup vs baseline: 3.0942x; 1.2140x over previous
import functools

import jax
import jax.numpy as jnp
from jax import lax
from jax.experimental import pallas as pl
from jax.experimental.pallas import tpu as pltpu
from jax.experimental.pallas import tpu_sc as plsc

N_NODES = 10000
N_EDGES = 160000
D = 256
DH = 128

NS = 16
BATCH = 128
NB = 80
E_PAD = NS * NB * BATCH
ACC_ROWS = 10240
ZROWS = ACC_ROWS // NS
WB_CHUNK = 128

RB = 1000


def _mm1_body(x_ref, w_ref, o_ref):
    h = jnp.dot(x_ref[...], w_ref[...], preferred_element_type=jnp.float32)
    o_ref[0] = h[:, :DH]
    o_ref[1] = h[:, DH:]


def _mm1(x, w):
    return pl.pallas_call(
        _mm1_body,
        grid=(N_NODES // RB,),
        in_specs=[
            pl.BlockSpec((RB, D), lambda r: (r, 0)),
            pl.BlockSpec((D, D), lambda r: (0, 0)),
        ],
        out_specs=pl.BlockSpec((2, RB, DH), lambda r: (0, r, 0)),
        out_shape=jax.ShapeDtypeStruct((2, N_NODES, DH), jnp.float32),
    )(x, w)


def _mm2_body(s_ref, b_ref, w_ref, o_ref):
    s = jnp.concatenate([s_ref[0], s_ref[1]], axis=1)
    a = jnp.maximum(s + b_ref[...], 0.0)
    h = jnp.dot(a, w_ref[...], preferred_element_type=jnp.float32)
    o_ref[0] = h[:, :DH]
    o_ref[1] = h[:, DH:]


def _mm2(s, b, w):
    return pl.pallas_call(
        _mm2_body,
        grid=(N_NODES // RB,),
        in_specs=[
            pl.BlockSpec((2, RB, DH), lambda r: (0, r, 0)),
            pl.BlockSpec((1, D), lambda r: (0, 0)),
            pl.BlockSpec((D, D), lambda r: (0, 0)),
        ],
        out_specs=pl.BlockSpec((2, RB, DH), lambda r: (0, r, 0)),
        out_shape=jax.ShapeDtypeStruct((2, N_NODES, DH), jnp.float32),
    )(s, b, w)


def _fin_body(s_ref, b_ref, o_ref):
    s = jnp.concatenate([s_ref[0], s_ref[1]], axis=1)
    o_ref[...] = jnp.maximum(s + b_ref[...], 0.0)


def _fin(s, b):
    return pl.pallas_call(
        _fin_body,
        grid=(N_NODES // RB,),
        in_specs=[
            pl.BlockSpec((2, RB, DH), lambda r: (0, r, 0)),
            pl.BlockSpec((1, D), lambda r: (0, 0)),
        ],
        out_specs=pl.BlockSpec((RB, D), lambda r: (r, 0)),
        out_shape=jax.ShapeDtypeStruct((N_NODES, D), jnp.float32),
    )(s, b)


def _sc_scatter_body(arr_hbm, h_hbm, z_hbm, o_hbm,
                     i0, i1, i2, i3, g0, g1, acc_sh,
                     is0, is1, is2, is3, gs0, gs1, ss0, ss1):
    c = lax.axis_index("c")
    s = lax.axis_index("s")
    ibufs = [i0, i1, i2, i3]
    isems = [is0, is1, is2, is3]
    gbufs = [g0, g1]
    gsems = [gs0, gs1]
    ssems = [ss0, ss1]

    arr_s = arr_hbm.at[s]
    h_c = h_hbm.at[c]

    def issue_idx(k, slot):
        pltpu.async_copy(arr_s.at[k], ibufs[slot], isems[slot])

    def wait_idx(slot):
        pltpu.make_async_copy(arr_s.at[0], ibufs[slot], isems[slot]).wait()

    def start_gather(jb, slot):
        pltpu.async_copy(h_c.at[ibufs[slot].at[0]], gbufs[jb], gsems[jb])

    def wait_gather(jb, slot):
        pltpu.make_async_copy(h_c.at[ibufs[slot].at[0]],
                              gbufs[jb], gsems[jb]).wait()

    def start_scatter(jb, slot):
        pltpu.async_copy(gbufs[jb], acc_sh.at[ibufs[slot].at[1]],
                         ssems[jb], add=True)

    def wait_scatter(jb, slot):
        pltpu.make_async_copy(gbufs[jb], acc_sh.at[ibufs[slot].at[1]],
                              ssems[jb]).wait()

    for t in range(4):
        issue_idx(t, t)
    pltpu.sync_copy(z_hbm, acc_sh.at[pl.ds(s * ZROWS, ZROWS)])
    plsc.subcore_barrier()
    wait_idx(0)
    start_gather(0, 0)

    @pl.loop(0, NB, step=4)
    def _edges(b):
        for j in range(4):
            jb = j % 2
            k = b + j
            wait_gather(jb, j)
            start_scatter(jb, j)

            @pl.when(k > 0)
            def _():
                wait_scatter(1 - jb, (j + 3) % 4)

            @pl.when(jnp.logical_and(k > 0, k + 3 < NB))
            def _():
                issue_idx(k + 3, (j + 3) % 4)

            @pl.when(k + 1 < NB)
            def _():
                wait_idx((j + 1) % 4)
                start_gather(1 - jb, (j + 1) % 4)

    wait_scatter((NB - 1) % 2, (NB - 1) % 4)
    plsc.subcore_barrier()

    o_c = o_hbm.at[c]

    @pl.loop(0, ZROWS // WB_CHUNK)
    def _wb(k):
        base = s * ZROWS + k * WB_CHUNK
        pltpu.sync_copy(acc_sh.at[pl.ds(base, WB_CHUNK)], g0)
        pltpu.sync_copy(g0, o_c.at[pl.ds(base, WB_CHUNK)])


@functools.cache
def _sc_scatter_kernel():
    mesh = plsc.VectorSubcoreMesh(core_axis_name="c", subcore_axis_name="s")
    return pl.kernel(
        _sc_scatter_body,
        out_type=jax.ShapeDtypeStruct((2, ACC_ROWS, DH), jnp.float32),
        mesh=mesh,
        scratch_types=[
            pltpu.VMEM((2, BATCH), jnp.int32),
            pltpu.VMEM((2, BATCH), jnp.int32),
            pltpu.VMEM((2, BATCH), jnp.int32),
            pltpu.VMEM((2, BATCH), jnp.int32),
            pltpu.VMEM((BATCH, DH), jnp.float32),
            pltpu.VMEM((BATCH, DH), jnp.float32),
            pltpu.VMEM_SHARED((ACC_ROWS, DH), jnp.float32),
        ] + [pltpu.SemaphoreType.DMA] * 8,
    )


def _sc_scatter(arr, h, zeros):
    return _sc_scatter_kernel()(arr, h, zeros)


def _prep_edges(edge_index):
    ei = edge_index.astype(jnp.int32)
    npad = E_PAD - N_EDGES
    col = jnp.concatenate([ei[1], jnp.zeros((npad,), jnp.int32)])
    row = jnp.concatenate([ei[0], jnp.full((npad,), N_NODES, jnp.int32)])
    return jnp.stack(
        [col.reshape(NS, NB, BATCH), row.reshape(NS, NB, BATCH)], axis=2)


def kernel(x, edge_index, W1, b1, W2, b2):
    arr = _prep_edges(edge_index)
    zeros = jnp.zeros((ZROWS, DH), jnp.float32)
    h1 = _mm1(x, W1)
    s1 = _sc_scatter(arr, h1, zeros)
    h2 = _mm2(s1, b1.reshape(1, D), W2)
    s2 = _sc_scatter(arr, h2, zeros)
    logits = _fin(s2, b2.reshape(1, D))
    return (logits, jnp.float32(0.0))
